# Initial kernel scaffold; baseline (speedup 1.0000x reference)
import jax
import jax.numpy as jnp
from jax.experimental import pallas as pl


def kernel(node_features, edge_node_indices, edge_features, W1e, b1e, W2e, b2e, W1n, b1n, W2n, b2n):
    raise NotImplementedError("write your pallas kernel here")


# speedup vs baseline: 7.5016x; 7.5016x over previous
import functools

import jax
import jax.numpy as jnp
from jax import lax
from jax.experimental import pallas as pl
from jax.experimental.pallas import tpu as pltpu
from jax.experimental.pallas import tpu_sc as plsc

_N = 10000
_E = 320000
_F = 128
_FE = 16
_H = 32
_MSG = 4

_NW = 32
_EW = _E // _NW
_GW = 80
_FOLD = 8
_E8 = _E // _FOLD
_CHUNK = 2000


def _tc_precompute(x, w1a, w1b):
    def body(x_ref, wa_ref, wb_ref, a_ref, b_ref):
        xv = x_ref[...]
        a_ref[...] = jnp.dot(xv, wa_ref[...], preferred_element_type=jnp.float32)
        b_ref[...] = jnp.dot(xv, wb_ref[...], preferred_element_type=jnp.float32)

    return pl.pallas_call(
        body,
        out_shape=[
            jax.ShapeDtypeStruct((_N, _H), jnp.float32),
            jax.ShapeDtypeStruct((_N, _H), jnp.float32),
        ],
    )(x, w1a, w1b)


def _sc_gather(a, b, n0, n1):
    mesh = plsc.VectorSubcoreMesh(core_axis_name="c", subcore_axis_name="s")

    @functools.partial(
        pl.kernel,
        out_type=jax.ShapeDtypeStruct((_E, _H), jnp.float32),
        mesh=mesh,
        compiler_params=pltpu.CompilerParams(use_tc_tiling_on_sc=False),
        scratch_types=[
            pltpu.VMEM((_EW,), jnp.int32),
            pltpu.VMEM((_EW,), jnp.int32),
            pltpu.VMEM((_GW, _H), jnp.float32),
            pltpu.VMEM((_GW, _H), jnp.float32),
            pltpu.SemaphoreType.DMA,
            pltpu.SemaphoreType.DMA,
        ],
    )
    def k(a_hbm, b_hbm, n0_hbm, n1_hbm, g_hbm, i0, i1, bufa, bufb, sa, sb):
        wid = lax.axis_index("s") * 2 + lax.axis_index("c")
        base = wid * _EW
        pltpu.sync_copy(n0_hbm.at[pl.ds(base, _EW)], i0)
        pltpu.sync_copy(n1_hbm.at[pl.ds(base, _EW)], i1)

        @pl.loop(0, _EW // _GW)
        def _(ci):
            off = ci * _GW
            ca = pltpu.async_copy(a_hbm.at[i0.at[pl.ds(off, _GW)]], bufa, sa)
            cb = pltpu.async_copy(b_hbm.at[i1.at[pl.ds(off, _GW)]], bufb, sb)
            ca.wait()
            cb.wait()

            @pl.loop(0, _GW)
            def _(r):
                bufa[r, pl.ds(0, 16)] += bufb[r, pl.ds(0, 16)]
                bufa[r, pl.ds(16, 16)] += bufb[r, pl.ds(16, 16)]

            pltpu.sync_copy(bufa, g_hbm.at[pl.ds(base + off, _GW)])

    return k(a, b, n0, n1)


def _tc_edge_mlp(g8, ef8, w1c8, b1e8, w2e8, b2e8):
    rows = 4000
    grid = (_E8 // rows,)

    def body(g_ref, e_ref, w1_ref, b1_ref, w2_ref, b2_ref, o_ref):
        h = (
            g_ref[...]
            + jnp.dot(e_ref[...], w1_ref[...], preferred_element_type=jnp.float32)
            + b1_ref[...]
        )
        h = h * jax.nn.sigmoid(h)
        m = jnp.dot(h, w2_ref[...], preferred_element_type=jnp.float32) + b2_ref[...]
        o_ref[...] = m * jax.nn.sigmoid(m)

    return pl.pallas_call(
        body,
        grid=grid,
        in_specs=[
            pl.BlockSpec((rows, _FOLD * _H), lambda i: (i, 0)),
            pl.BlockSpec((rows, _FOLD * _FE), lambda i: (i, 0)),
            pl.BlockSpec((_FOLD * _FE, _FOLD * _H), lambda i: (0, 0)),
            pl.BlockSpec((1, _FOLD * _H), lambda i: (0, 0)),
            pl.BlockSpec((_FOLD * _H, _FOLD * _MSG), lambda i: (0, 0)),
            pl.BlockSpec((1, _FOLD * _MSG), lambda i: (0, 0)),
        ],
        out_specs=pl.BlockSpec((rows, _FOLD * _MSG), lambda i: (i, 0)),
        out_shape=jax.ShapeDtypeStruct((_E8, _FOLD * _MSG), jnp.float32),
    )(g8, ef8, w1c8, b1e8, w2e8, b2e8)


def _sc_scatter(n0, m_flat):
    mesh = plsc.VectorSubcoreMesh(core_axis_name="c", subcore_axis_name="s")

    @functools.partial(
        pl.kernel,
        out_type=jax.ShapeDtypeStruct((_MSG * _NW, _N), jnp.float32),
        mesh=mesh,
        compiler_params=pltpu.CompilerParams(
            use_tc_tiling_on_sc=False, needs_layout_passes=False),
        scratch_types=[
            pltpu.VMEM((_MSG, _N), jnp.float32),
            pltpu.VMEM((_CHUNK,), jnp.int32),
            pltpu.VMEM((_CHUNK * _MSG,), jnp.float32),
        ],
    )
    def k(n0_hbm, m_hbm, o_hbm, acc, idxb, mb):
        wid = lax.axis_index("s") * 2 + lax.axis_index("c")

        @pl.loop(0, _N, step=16)
        def _(j):
            z = jnp.zeros((16,), jnp.float32)
            acc[0, pl.ds(j, 16)] = z
            acc[1, pl.ds(j, 16)] = z
            acc[2, pl.ds(j, 16)] = z
            acc[3, pl.ds(j, 16)] = z

        lanes = lax.broadcasted_iota(jnp.int32, (16,), 0)
        e_rep = lanes >> 2
        k_rep = lanes & 3

        @pl.loop(0, _EW // _CHUNK)
        def _(ci):
            base = wid * _EW + ci * _CHUNK
            pltpu.sync_copy(n0_hbm.at[pl.ds(base, _CHUNK)], idxb)
            pltpu.sync_copy(m_hbm.at[pl.ds(base * _MSG, _CHUNK * _MSG)], mb)

            @pl.loop(0, _CHUNK, step=4)
            def _(e0):
                nvec = plsc.load_gather(idxb, [e0 + e_rep])
                vals = mb[pl.ds(e0 * _MSG, 16)]
                plsc.addupdate_scatter(acc, [k_rep, nvec], vals, mask=e_rep == 0)
                plsc.addupdate_scatter(acc, [k_rep, nvec], vals, mask=e_rep == 1)
                plsc.addupdate_scatter(acc, [k_rep, nvec], vals, mask=e_rep == 2)
                plsc.addupdate_scatter(acc, [k_rep, nvec], vals, mask=e_rep == 3)

        pltpu.sync_copy(acc, o_hbm.at[pl.ds(_MSG * wid, _MSG)])

    return k(n0, m_flat)


def _tc_node_mlp(x, pt, w1nx, w1nmt, b1n, w2n, b2n):
    def body(x_ref, p_ref, wx_ref, wm_ref, b1_ref, w2_ref, b2_ref, o_ref):
        h = (
            jnp.dot(x_ref[...], wx_ref[...], preferred_element_type=jnp.float32)
            + lax.dot_general(
                p_ref[...], wm_ref[...], (((0,), (0,)), ((), ())),
                preferred_element_type=jnp.float32,
            )
            + b1_ref[...]
        )
        h = h * jax.nn.sigmoid(h)
        o = jnp.dot(h, w2_ref[...], preferred_element_type=jnp.float32) + b2_ref[...]
        o_ref[...] = o * jax.nn.sigmoid(o)

    return pl.pallas_call(
        body,
        out_shape=jax.ShapeDtypeStruct((_N, _F), jnp.float32),
    )(x, pt, w1nx, w1nmt, b1n, w2n, b2n)


def kernel(node_features, edge_node_indices, edge_features,
           W1e, b1e, W2e, b2e, W1n, b1n, W2n, b2n):
    n0 = edge_node_indices[0]
    n1 = edge_node_indices[1]
    w1a, w1b, w1c = W1e[:_F], W1e[_F:2 * _F], W1e[2 * _F:]

    a, b = _tc_precompute(node_features, w1a, w1b)
    g = _sc_gather(a, b, n0, n1)

    eye = jnp.eye(_FOLD, dtype=jnp.float32)
    w1c8 = jnp.kron(eye, w1c)
    b1e8 = jnp.tile(b1e, _FOLD).reshape(1, -1)
    w2e8 = jnp.kron(eye, W2e)
    b2e8 = jnp.tile(b2e, _FOLD).reshape(1, -1)
    g8 = g.reshape(_E8, _FOLD * _H)
    ef8 = edge_features.reshape(_E8, _FOLD * _FE)
    m8 = _tc_edge_mlp(g8, ef8, w1c8, b1e8, w2e8, b2e8)

    pt = _sc_scatter(n0, m8.reshape(-1))

    w1nx, w1nm = W1n[:_F], W1n[_F:]
    w1nmt = jnp.tile(w1nm, (_NW, 1))
    return _tc_node_mlp(node_features, pt, w1nx, w1nmt,
                        b1n.reshape(1, -1), W2n, b2n.reshape(1, -1))


# speedup vs baseline: 8.9559x; 1.1939x over previous
import functools

import jax
import jax.numpy as jnp
from jax import lax
from jax.experimental import pallas as pl
from jax.experimental.pallas import tpu as pltpu
from jax.experimental.pallas import tpu_sc as plsc

_N = 10000
_E = 320000
_F = 128
_FE = 16
_H = 32
_MSG = 4

_NW = 32
_EW = _E // _NW
_GW = 80
_FOLD = 8
_E8 = _E // _FOLD
_CHUNK = 2000


def _tc_precompute(x, w1a, w1b):
    def body(x_ref, wa_ref, wb_ref, a_ref, b_ref):
        xv = x_ref[...]
        a_ref[...] = jnp.dot(xv, wa_ref[...], preferred_element_type=jnp.float32)
        b_ref[...] = jnp.dot(xv, wb_ref[...], preferred_element_type=jnp.float32)

    return pl.pallas_call(
        body,
        out_shape=[
            jax.ShapeDtypeStruct((_N, _H), jnp.float32),
            jax.ShapeDtypeStruct((_N, _H), jnp.float32),
        ],
    )(x, w1a, w1b)


def _sc_gather(a, b, n0, n1):
    mesh = plsc.VectorSubcoreMesh(core_axis_name="c", subcore_axis_name="s")

    nwin = _EW // _GW

    @functools.partial(
        pl.kernel,
        out_type=jax.ShapeDtypeStruct((_E, _H), jnp.float32),
        mesh=mesh,
        compiler_params=pltpu.CompilerParams(use_tc_tiling_on_sc=False),
        scratch_types=[
            pltpu.VMEM((_EW,), jnp.int32),
            pltpu.VMEM((_EW,), jnp.int32),
            pltpu.VMEM((_GW, _H), jnp.float32),
            pltpu.VMEM((_GW, _H), jnp.float32),
            pltpu.VMEM((_GW, _H), jnp.float32),
            pltpu.VMEM((_GW, _H), jnp.float32),
            pltpu.VMEM((_GW, _H), jnp.float32),
            pltpu.VMEM((_GW, _H), jnp.float32),
            pltpu.SemaphoreType.DMA,
            pltpu.SemaphoreType.DMA,
            pltpu.SemaphoreType.DMA,
            pltpu.SemaphoreType.DMA,
            pltpu.SemaphoreType.DMA,
            pltpu.SemaphoreType.DMA,
        ],
    )
    def k(a_hbm, b_hbm, n0_hbm, n1_hbm, g_hbm, i0, i1,
          a0, b0, a1, b1, o0, o1, sa0, sb0, sa1, sb1, w0, w1):
        wid = lax.axis_index("s") * 2 + lax.axis_index("c")
        base = wid * _EW
        pltpu.sync_copy(n0_hbm.at[pl.ds(base, _EW)], i0)
        pltpu.sync_copy(n1_hbm.at[pl.ds(base, _EW)], i1)

        def issue(w, abuf, bbuf, sa, sb):
            off = w * _GW
            pltpu.async_copy(a_hbm.at[i0.at[pl.ds(off, _GW)]], abuf, sa)
            pltpu.async_copy(b_hbm.at[i1.at[pl.ds(off, _GW)]], bbuf, sb)

        def wait_gathers(abuf, bbuf, sa, sb):
            pltpu.make_async_copy(a_hbm.at[i0.at[pl.ds(0, _GW)]], abuf, sa).wait()
            pltpu.make_async_copy(b_hbm.at[i1.at[pl.ds(0, _GW)]], bbuf, sb).wait()

        def add_rows(abuf, bbuf, obuf):
            @pl.loop(0, _GW)
            def _(r):
                obuf[r, pl.ds(0, 16)] = abuf[r, pl.ds(0, 16)] + bbuf[r, pl.ds(0, 16)]
                obuf[r, pl.ds(16, 16)] = abuf[r, pl.ds(16, 16)] + bbuf[r, pl.ds(16, 16)]

        def wait_write(obuf, ws):
            pltpu.make_async_copy(obuf, g_hbm.at[pl.ds(base, _GW)], ws).wait()

        issue(0, a0, b0, sa0, sb0)
        issue(1, a1, b1, sa1, sb1)

        @pl.loop(0, nwin - 1, step=2)
        def _(ci):
            wait_gathers(a0, b0, sa0, sb0)

            @pl.when(ci >= 2)
            def _():
                wait_write(o0, w0)

            add_rows(a0, b0, o0)
            pltpu.async_copy(o0, g_hbm.at[pl.ds(base + ci * _GW, _GW)], w0)
            issue(ci + 2, a0, b0, sa0, sb0)

            wait_gathers(a1, b1, sa1, sb1)

            @pl.when(ci >= 2)
            def _():
                wait_write(o1, w1)

            add_rows(a1, b1, o1)
            pltpu.async_copy(o1, g_hbm.at[pl.ds(base + (ci + 1) * _GW, _GW)], w1)

            @pl.when(ci + 3 < nwin)
            def _():
                issue(ci + 3, a1, b1, sa1, sb1)

        wait_gathers(a0, b0, sa0, sb0)
        wait_write(o0, w0)
        add_rows(a0, b0, o0)
        pltpu.async_copy(o0, g_hbm.at[pl.ds(base + (nwin - 1) * _GW, _GW)], w0)
        wait_write(o0, w0)
        wait_write(o1, w1)

    return k(a, b, n0, n1)


def _tc_edge_mlp(g8, ef8, w1c8, b1e8, w2e8, b2e8):
    rows = 4000
    grid = (_E8 // rows,)

    def body(g_ref, e_ref, w1_ref, b1_ref, w2_ref, b2_ref, o_ref):
        h = (
            g_ref[...]
            + jnp.dot(e_ref[...], w1_ref[...], preferred_element_type=jnp.float32)
            + b1_ref[...]
        )
        h = h * jax.nn.sigmoid(h)
        m = jnp.dot(h, w2_ref[...], preferred_element_type=jnp.float32) + b2_ref[...]
        o_ref[...] = m * jax.nn.sigmoid(m)

    return pl.pallas_call(
        body,
        grid=grid,
        in_specs=[
            pl.BlockSpec((rows, _FOLD * _H), lambda i: (i, 0)),
            pl.BlockSpec((rows, _FOLD * _FE), lambda i: (i, 0)),
            pl.BlockSpec((_FOLD * _FE, _FOLD * _H), lambda i: (0, 0)),
            pl.BlockSpec((1, _FOLD * _H), lambda i: (0, 0)),
            pl.BlockSpec((_FOLD * _H, _FOLD * _MSG), lambda i: (0, 0)),
            pl.BlockSpec((1, _FOLD * _MSG), lambda i: (0, 0)),
        ],
        out_specs=pl.BlockSpec((rows, _FOLD * _MSG), lambda i: (i, 0)),
        out_shape=jax.ShapeDtypeStruct((_E8, _FOLD * _MSG), jnp.float32),
    )(g8, ef8, w1c8, b1e8, w2e8, b2e8)


def _sc_scatter(n0, m_flat):
    mesh = plsc.VectorSubcoreMesh(core_axis_name="c", subcore_axis_name="s")

    @functools.partial(
        pl.kernel,
        out_type=jax.ShapeDtypeStruct((_MSG * _NW, _N), jnp.float32),
        mesh=mesh,
        compiler_params=pltpu.CompilerParams(
            use_tc_tiling_on_sc=False, needs_layout_passes=False),
        scratch_types=[
            pltpu.VMEM((_MSG, _N), jnp.float32),
            pltpu.VMEM((_CHUNK,), jnp.int32),
            pltpu.VMEM((_CHUNK * _MSG,), jnp.float32),
        ],
    )
    def k(n0_hbm, m_hbm, o_hbm, acc, idxb, mb):
        wid = lax.axis_index("s") * 2 + lax.axis_index("c")

        @pl.loop(0, _N, step=16)
        def _(j):
            z = jnp.zeros((16,), jnp.float32)
            acc[0, pl.ds(j, 16)] = z
            acc[1, pl.ds(j, 16)] = z
            acc[2, pl.ds(j, 16)] = z
            acc[3, pl.ds(j, 16)] = z

        lanes = lax.broadcasted_iota(jnp.int32, (16,), 0)
        e_rep = lanes >> 2
        k_rep = lanes & 3

        @pl.loop(0, _EW // _CHUNK)
        def _(ci):
            base = wid * _EW + ci * _CHUNK
            pltpu.sync_copy(n0_hbm.at[pl.ds(base, _CHUNK)], idxb)
            pltpu.sync_copy(m_hbm.at[pl.ds(base * _MSG, _CHUNK * _MSG)], mb)

            @pl.loop(0, _CHUNK, step=4)
            def _(e0):
                nvec = plsc.load_gather(idxb, [e0 + e_rep])
                vals = mb[pl.ds(e0 * _MSG, 16)]
                plsc.addupdate_scatter(acc, [k_rep, nvec], vals, mask=e_rep == 0)
                plsc.addupdate_scatter(acc, [k_rep, nvec], vals, mask=e_rep == 1)
                plsc.addupdate_scatter(acc, [k_rep, nvec], vals, mask=e_rep == 2)
                plsc.addupdate_scatter(acc, [k_rep, nvec], vals, mask=e_rep == 3)

        pltpu.sync_copy(acc, o_hbm.at[pl.ds(_MSG * wid, _MSG)])

    return k(n0, m_flat)


def _tc_node_mlp(x, pt, w1nx, w1nmt, b1n, w2n, b2n):
    def body(x_ref, p_ref, wx_ref, wm_ref, b1_ref, w2_ref, b2_ref, o_ref):
        h = (
            jnp.dot(x_ref[...], wx_ref[...], preferred_element_type=jnp.float32)
            + lax.dot_general(
                p_ref[...], wm_ref[...], (((0,), (0,)), ((), ())),
                preferred_element_type=jnp.float32,
            )
            + b1_ref[...]
        )
        h = h * jax.nn.sigmoid(h)
        o = jnp.dot(h, w2_ref[...], preferred_element_type=jnp.float32) + b2_ref[...]
        o_ref[...] = o * jax.nn.sigmoid(o)

    return pl.pallas_call(
        body,
        out_shape=jax.ShapeDtypeStruct((_N, _F), jnp.float32),
    )(x, pt, w1nx, w1nmt, b1n, w2n, b2n)


def kernel(node_features, edge_node_indices, edge_features,
           W1e, b1e, W2e, b2e, W1n, b1n, W2n, b2n):
    n0 = edge_node_indices[0]
    n1 = edge_node_indices[1]
    w1a, w1b, w1c = W1e[:_F], W1e[_F:2 * _F], W1e[2 * _F:]

    a, b = _tc_precompute(node_features, w1a, w1b)
    g = _sc_gather(a, b, n0, n1)

    eye = jnp.eye(_FOLD, dtype=jnp.float32)
    w1c8 = jnp.kron(eye, w1c)
    b1e8 = jnp.tile(b1e, _FOLD).reshape(1, -1)
    w2e8 = jnp.kron(eye, W2e)
    b2e8 = jnp.tile(b2e, _FOLD).reshape(1, -1)
    g8 = g.reshape(_E8, _FOLD * _H)
    ef8 = edge_features.reshape(_E8, _FOLD * _FE)
    m8 = _tc_edge_mlp(g8, ef8, w1c8, b1e8, w2e8, b2e8)

    pt = _sc_scatter(n0, m8.reshape(-1))

    w1nx, w1nm = W1n[:_F], W1n[_F:]
    w1nmt = jnp.tile(w1nm, (_NW, 1))
    return _tc_node_mlp(node_features, pt, w1nx, w1nmt,
                        b1n.reshape(1, -1), W2n, b2n.reshape(1, -1))


# speedup vs baseline: 9.2539x; 1.0333x over previous
import functools

import jax
import jax.numpy as jnp
from jax import lax
from jax.experimental import pallas as pl
from jax.experimental.pallas import tpu as pltpu
from jax.experimental.pallas import tpu_sc as plsc

_N = 10000
_E = 320000
_F = 128
_FE = 16
_H = 32
_MSG = 4

_NW = 32
_EW = _E // _NW
_GW = 80
_FOLD = 4
_E4 = _E // _FOLD
_CHUNK = 2000


def _tc_precompute(x, w1a, w1b):
    def body(x_ref, wa_ref, wb_ref, a_ref, b_ref):
        xv = x_ref[...]
        a_ref[...] = jnp.dot(xv, wa_ref[...], preferred_element_type=jnp.float32)
        b_ref[...] = jnp.dot(xv, wb_ref[...], preferred_element_type=jnp.float32)

    return pl.pallas_call(
        body,
        out_shape=[
            jax.ShapeDtypeStruct((_N, _H), jnp.float32),
            jax.ShapeDtypeStruct((_N, _H), jnp.float32),
        ],
    )(x, w1a, w1b)


def _sc_gather(a, b, n0, n1):
    mesh = plsc.VectorSubcoreMesh(core_axis_name="c", subcore_axis_name="s")

    nwin = _EW // _GW
    grows = _GW // 4

    @functools.partial(
        pl.kernel,
        out_type=jax.ShapeDtypeStruct((_E4, 4 * _H), jnp.float32),
        mesh=mesh,
        compiler_params=pltpu.CompilerParams(use_tc_tiling_on_sc=False),
        scratch_types=[
            pltpu.VMEM((_EW,), jnp.int32),
            pltpu.VMEM((_EW,), jnp.int32),
            pltpu.VMEM((_GW, _H), jnp.float32),
            pltpu.VMEM((_GW, _H), jnp.float32),
            pltpu.VMEM((_GW, _H), jnp.float32),
            pltpu.VMEM((_GW, _H), jnp.float32),
            pltpu.VMEM((grows, 4 * _H), jnp.float32),
            pltpu.VMEM((grows, 4 * _H), jnp.float32),
            pltpu.SemaphoreType.DMA,
            pltpu.SemaphoreType.DMA,
            pltpu.SemaphoreType.DMA,
            pltpu.SemaphoreType.DMA,
            pltpu.SemaphoreType.DMA,
            pltpu.SemaphoreType.DMA,
        ],
    )
    def k(a_hbm, b_hbm, n0_hbm, n1_hbm, g_hbm, i0, i1,
          a0, b0, a1, b1, o0, o1, sa0, sb0, sa1, sb1, w0, w1):
        wid = lax.axis_index("s") * 2 + lax.axis_index("c")
        base = wid * _EW
        base4 = wid * (_EW // 4)
        pltpu.sync_copy(n0_hbm.at[pl.ds(base, _EW)], i0)
        pltpu.sync_copy(n1_hbm.at[pl.ds(base, _EW)], i1)

        def issue(w, abuf, bbuf, sa, sb):
            off = w * _GW
            pltpu.async_copy(a_hbm.at[i0.at[pl.ds(off, _GW)]], abuf, sa)
            pltpu.async_copy(b_hbm.at[i1.at[pl.ds(off, _GW)]], bbuf, sb)

        def wait_gathers(abuf, bbuf, sa, sb):
            pltpu.make_async_copy(a_hbm.at[i0.at[pl.ds(0, _GW)]], abuf, sa).wait()
            pltpu.make_async_copy(b_hbm.at[i1.at[pl.ds(0, _GW)]], bbuf, sb).wait()

        def add_rows(abuf, bbuf, obuf):
            @pl.loop(0, grows)
            def _(q):
                for s in range(4):
                    for h0 in (0, 16):
                        obuf[q, pl.ds(s * _H + h0, 16)] = (
                            abuf[4 * q + s, pl.ds(h0, 16)]
                            + bbuf[4 * q + s, pl.ds(h0, 16)]
                        )

        def wait_write(obuf, ws):
            pltpu.make_async_copy(obuf, g_hbm.at[pl.ds(0, grows)], ws).wait()

        issue(0, a0, b0, sa0, sb0)
        issue(1, a1, b1, sa1, sb1)

        @pl.loop(0, nwin - 1, step=2)
        def _(ci):
            wait_gathers(a0, b0, sa0, sb0)

            @pl.when(ci >= 2)
            def _():
                wait_write(o0, w0)

            add_rows(a0, b0, o0)
            pltpu.async_copy(o0, g_hbm.at[pl.ds(base4 + ci * grows, grows)], w0)
            issue(ci + 2, a0, b0, sa0, sb0)

            wait_gathers(a1, b1, sa1, sb1)

            @pl.when(ci >= 2)
            def _():
                wait_write(o1, w1)

            add_rows(a1, b1, o1)
            pltpu.async_copy(o1, g_hbm.at[pl.ds(base4 + (ci + 1) * grows, grows)], w1)

            @pl.when(ci + 3 < nwin)
            def _():
                issue(ci + 3, a1, b1, sa1, sb1)

        wait_gathers(a0, b0, sa0, sb0)
        wait_write(o0, w0)
        add_rows(a0, b0, o0)
        pltpu.async_copy(o0, g_hbm.at[pl.ds(base4 + (nwin - 1) * grows, grows)], w0)
        wait_write(o0, w0)
        wait_write(o1, w1)

    return k(a, b, n0, n1)


def _tc_edge_mlp(g4, ef4, w1c4, b1e4, w2e4, b2e4):
    rows = 8000
    grid = (_E4 // rows,)

    def body(g_ref, e_ref, w1_ref, b1_ref, w2_ref, b2_ref, o_ref):
        h = (
            g_ref[...]
            + jnp.dot(e_ref[...], w1_ref[...], preferred_element_type=jnp.float32)
            + b1_ref[...]
        )
        h = h * jax.nn.sigmoid(h)
        m = jnp.dot(h, w2_ref[...], preferred_element_type=jnp.float32) + b2_ref[...]
        o_ref[...] = m * jax.nn.sigmoid(m)

    return pl.pallas_call(
        body,
        grid=grid,
        in_specs=[
            pl.BlockSpec((rows, _FOLD * _H), lambda i: (i, 0)),
            pl.BlockSpec((rows, _FOLD * _FE), lambda i: (i, 0)),
            pl.BlockSpec((_FOLD * _FE, _FOLD * _H), lambda i: (0, 0)),
            pl.BlockSpec((1, _FOLD * _H), lambda i: (0, 0)),
            pl.BlockSpec((_FOLD * _H, _FOLD * _MSG), lambda i: (0, 0)),
            pl.BlockSpec((1, _FOLD * _MSG), lambda i: (0, 0)),
        ],
        out_specs=pl.BlockSpec((rows, _FOLD * _MSG), lambda i: (i, 0)),
        out_shape=jax.ShapeDtypeStruct((_E4, _FOLD * _MSG), jnp.float32),
    )(g4, ef4, w1c4, b1e4, w2e4, b2e4)


def _sc_scatter(n0, m_flat):
    mesh = plsc.VectorSubcoreMesh(core_axis_name="c", subcore_axis_name="s")

    @functools.partial(
        pl.kernel,
        out_type=jax.ShapeDtypeStruct((_MSG * _NW, _N), jnp.float32),
        mesh=mesh,
        compiler_params=pltpu.CompilerParams(
            use_tc_tiling_on_sc=False, needs_layout_passes=False),
        scratch_types=[
            pltpu.VMEM((_MSG, _N), jnp.float32),
            pltpu.VMEM((_CHUNK,), jnp.int32),
            pltpu.VMEM((_CHUNK * _MSG,), jnp.float32),
        ],
    )
    def k(n0_hbm, m_hbm, o_hbm, acc, idxb, mb):
        wid = lax.axis_index("s") * 2 + lax.axis_index("c")

        @pl.loop(0, _N, step=16)
        def _(j):
            z = jnp.zeros((16,), jnp.float32)
            acc[0, pl.ds(j, 16)] = z
            acc[1, pl.ds(j, 16)] = z
            acc[2, pl.ds(j, 16)] = z
            acc[3, pl.ds(j, 16)] = z

        lanes = lax.broadcasted_iota(jnp.int32, (16,), 0)
        e_rep = lanes >> 2
        k_rep = lanes & 3

        @pl.loop(0, _EW // _CHUNK)
        def _(ci):
            base = wid * _EW + ci * _CHUNK
            pltpu.sync_copy(n0_hbm.at[pl.ds(base, _CHUNK)], idxb)
            pltpu.sync_copy(m_hbm.at[pl.ds(base * _MSG, _CHUNK * _MSG)], mb)

            @pl.loop(0, _CHUNK, step=4)
            def _(e0):
                nvec = plsc.load_gather(idxb, [e0 + e_rep])
                vals = mb[pl.ds(e0 * _MSG, 16)]
                plsc.addupdate_scatter(acc, [k_rep, nvec], vals, mask=e_rep == 0)
                plsc.addupdate_scatter(acc, [k_rep, nvec], vals, mask=e_rep == 1)
                plsc.addupdate_scatter(acc, [k_rep, nvec], vals, mask=e_rep == 2)
                plsc.addupdate_scatter(acc, [k_rep, nvec], vals, mask=e_rep == 3)

        pltpu.sync_copy(acc, o_hbm.at[pl.ds(_MSG * wid, _MSG)])

    return k(n0, m_flat)


def _tc_node_mlp(x, pt, w1nx, w1nmt, b1n, w2n, b2n):
    def body(x_ref, p_ref, wx_ref, wm_ref, b1_ref, w2_ref, b2_ref, o_ref):
        h = (
            jnp.dot(x_ref[...], wx_ref[...], preferred_element_type=jnp.float32)
            + lax.dot_general(
                p_ref[...], wm_ref[...], (((0,), (0,)), ((), ())),
                preferred_element_type=jnp.float32,
            )
            + b1_ref[...]
        )
        h = h * jax.nn.sigmoid(h)
        o = jnp.dot(h, w2_ref[...], preferred_element_type=jnp.float32) + b2_ref[...]
        o_ref[...] = o * jax.nn.sigmoid(o)

    return pl.pallas_call(
        body,
        out_shape=jax.ShapeDtypeStruct((_N, _F), jnp.float32),
    )(x, pt, w1nx, w1nmt, b1n, w2n, b2n)


def kernel(node_features, edge_node_indices, edge_features,
           W1e, b1e, W2e, b2e, W1n, b1n, W2n, b2n):
    n0 = edge_node_indices[0]
    n1 = edge_node_indices[1]
    w1a, w1b, w1c = W1e[:_F], W1e[_F:2 * _F], W1e[2 * _F:]

    a, b = _tc_precompute(node_features, w1a, w1b)
    g4 = _sc_gather(a, b, n0, n1)

    eye = jnp.eye(_FOLD, dtype=jnp.float32)
    w1c4 = jnp.kron(eye, w1c)
    b1e4 = jnp.tile(b1e, _FOLD).reshape(1, -1)
    w2e4 = jnp.kron(eye, W2e)
    b2e4 = jnp.tile(b2e, _FOLD).reshape(1, -1)
    ef4 = edge_features.reshape(_E4, _FOLD * _FE)
    m4 = _tc_edge_mlp(g4, ef4, w1c4, b1e4, w2e4, b2e4)

    pt = _sc_scatter(n0, m4.reshape(-1))

    w1nx, w1nm = W1n[:_F], W1n[_F:]
    w1nmt = jnp.tile(w1nm, (_NW, 1))
    return _tc_node_mlp(node_features, pt, w1nx, w1nmt,
                        b1n.reshape(1, -1), W2n, b2n.reshape(1, -1))
